# speedup vs baseline: 2.2212x; 2.2212x over previous
import functools

import jax
import jax.numpy as jnp
from jax.experimental import pallas as pl
from jax.experimental.pallas import tpu as pltpu


def _router_body(x_ref, w_ref, probs_ref, map_ref):
    x = x_ref[...]
    w = w_ref[...]
    logits = jax.lax.dot_general(
        x, w, (((1,), (1,)), ((), ())), preferred_element_type=jnp.float32
    )
    tt, e = logits.shape
    iota = jax.lax.broadcasted_iota(jnp.int32, (tt, e), 1)
    m1 = jnp.max(logits, axis=1, keepdims=True)
    idx1 = jnp.min(jnp.where(logits == m1, iota, e), axis=1, keepdims=True)
    masked = jnp.where(iota == idx1, -jnp.inf, logits)
    m2 = jnp.max(masked, axis=1, keepdims=True)
    idx2 = jnp.min(jnp.where(masked == m2, iota, e), axis=1, keepdims=True)
    t = jnp.exp(m2 - m1)
    denom = 1.0 + t
    p1 = 1.0 / denom
    p2 = t / denom
    probs_ref[...] = jnp.where(
        iota == idx1, p1, jnp.where(iota == idx2, p2, 0.0)
    )
    map_ref[...] = (iota == idx1) | (iota == idx2)


@functools.partial(jax.jit, static_argnames=("tt",))
def _route(x, w, tt):
    tokens, d = x.shape
    e = w.shape[0]
    grid = (tokens // tt,)
    return pl.pallas_call(
        _router_body,
        grid=grid,
        in_specs=[
            pl.BlockSpec((tt, d), lambda i: (i, 0)),
            pl.BlockSpec((e, d), lambda i: (0, 0)),
        ],
        out_specs=[
            pl.BlockSpec((tt, e), lambda i: (i, 0)),
            pl.BlockSpec((tt, e), lambda i: (i, 0)),
        ],
        out_shape=[
            jax.ShapeDtypeStruct((tokens, e), jnp.float32),
            jax.ShapeDtypeStruct((tokens, e), jnp.bool_),
        ],
        compiler_params=pltpu.CompilerParams(
            dimension_semantics=("arbitrary",),
        ),
    )(x, w)


def kernel(hidden_states, router_weight):
    s, b, d = hidden_states.shape
    x = hidden_states.reshape(s * b, d).astype(jnp.float32)
    probs, routing_map = _route(x, router_weight.astype(jnp.float32), tt=1024)
    return probs, routing_map
